# baseline (device time: 53399 ns/iter reference)
import jax
import jax.numpy as jnp
from jax import lax
from jax.experimental import pallas as pl
from jax.experimental.pallas import tpu as pltpu

N_DEV = 4
B, SQ, D = 4, 256, 1024
HQ_SH = 8
HKV_SH = 2
DH = 128
GROUP = 4
SCALE = 0.08838834764831843
BS = B * SQ
HSQ = SQ // 2
QSQ = HSQ // 2


def kernel(x, Wq, Wo, Wk, Wv):
    kv_off = lax.axis_index("i") * (HKV_SH * DH)
    Wk_sh = lax.dynamic_slice_in_dim(Wk, kv_off, HKV_SH * DH, axis=1)
    Wv_sh = lax.dynamic_slice_in_dim(Wv, kv_off, HKV_SH * DH, axis=1)

    def body(x_ref, wq_ref, wo_ref, wk_ref, wv_ref, out_ref,
             pbuf, s1p_buf, s1q_buf, s2p_send, s2p_buf, s2q_send, s2q_buf,
             redp_buf, redq_buf, s3p_buf, s3q_buf, s4p_buf, s4q_buf,
             send_sems, recv_sems):
        my_i = lax.axis_index("i")
        p1 = my_i ^ 1
        p2 = 3 - my_i
        diag = (my_i + 2) % N_DEV

        barrier_sem = pltpu.get_barrier_semaphore()
        for o in range(1, N_DEV):
            pl.semaphore_signal(barrier_sem, inc=1,
                                device_id=((my_i + o) % N_DEV,),
                                device_id_type=pl.DeviceIdType.MESH)
        pl.semaphore_wait(barrier_sem, N_DEV - 1)

        q1 = pl.ds(0, QSQ)
        q2 = pl.ds(QSQ, QSQ)

        transfers = [
            (pbuf.at[0, 0], s1p_buf.at[0], p1),
            (pbuf.at[0, 1], s1q_buf.at[0], p2),
            (pbuf.at[1, 0], s1p_buf.at[1], p1),
            (pbuf.at[2, 1], s1q_buf.at[1], p2),
            (s2p_send, s2p_buf, p2),
            (s2q_send, s2q_buf, p1),
            (redp_buf.at[q1], s3p_buf.at[q1], p2),
            (redp_buf.at[q2], s3p_buf.at[q2], p2),
            (redq_buf.at[q1], s3q_buf.at[q1], p1),
            (redq_buf.at[q2], s3q_buf.at[q2], p1),
            (redp_buf, s4p_buf.at[0], p1),
            (redq_buf, s4q_buf.at[0], p2),
            (s3p_buf.at[q1], s4p_buf.at[1, q1], p1),
            (s3p_buf.at[q2], s4p_buf.at[1, q2], p1),
            (s3q_buf.at[q1], s4q_buf.at[1, q1], p2),
            (s3q_buf.at[q2], s4q_buf.at[1, q2], p2),
        ]
        d = [
            pltpu.make_async_remote_copy(
                src_ref=src, dst_ref=dst,
                send_sem=send_sems.at[k], recv_sem=recv_sems.at[k],
                device_id=(tgt,), device_id_type=pl.DeviceIdType.MESH,
            )
            for k, (src, dst, tgt) in enumerate(transfers)
        ]

        wq = wq_ref[...].astype(jnp.bfloat16)
        wk = wk_ref[...].astype(jnp.bfloat16)
        wv = wv_ref[...].astype(jnp.bfloat16)
        wo = wo_ref[...].astype(jnp.bfloat16)

        def store_partial(c, b):
            xb = x_ref[b].astype(jnp.bfloat16)
            qb = jnp.dot(xb, wq, preferred_element_type=jnp.float32)
            kb = jnp.dot(xb, wk, preferred_element_type=jnp.float32)
            vb = jnp.dot(xb, wv, preferred_element_type=jnp.float32)
            outs = []
            for h in range(HQ_SH):
                g = h // GROUP
                q = qb[:, h * DH:(h + 1) * DH]
                k = kb[:, g * DH:(g + 1) * DH]
                v = vb[:, g * DH:(g + 1) * DH]
                s = jnp.dot(q, k.T, preferred_element_type=jnp.float32) * SCALE
                m = jnp.max(s, axis=-1, keepdims=True)
                p = jnp.exp(s - m)
                l = jnp.sum(p, axis=-1, keepdims=True)
                outs.append(jnp.dot(p, v, preferred_element_type=jnp.float32) / l)
            attn_b = jnp.concatenate(outs, axis=1).astype(jnp.bfloat16)
            partial = jnp.dot(attn_b, wo, preferred_element_type=jnp.float32)
            pbuf[c, 0] = partial[:HSQ, :]
            pbuf[c, 1] = partial[HSQ:, :]

        def out_rows(b, half):
            return pl.ds(b * SQ + half * HSQ, HSQ)

        store_partial(0, diag)
        d[0].start()
        d[1].start()
        store_partial(1, p1)
        d[2].start()
        store_partial(2, p2)
        d[3].start()

        d[0].wait_recv()
        s2p_send[...] = pbuf[2, 0] + s1p_buf[0]
        d[4].start()
        d[1].wait_recv()
        s2q_send[...] = pbuf[1, 1] + s1q_buf[0]
        d[5].start()

        store_partial(3, my_i)

        d[3].wait_recv()
        d[5].wait_recv()
        redq_buf[...] = pbuf[3, 1] + s1q_buf[1] + s2q_buf[...]
        d[8].start()
        d[9].start()
        d[11].start()
        d[2].wait_recv()
        d[4].wait_recv()
        redp_buf[...] = pbuf[3, 0] + s1p_buf[1] + s2p_buf[...]
        d[6].start()
        d[7].start()
        d[10].start()
        out_ref[out_rows(my_i, 0), :] = redp_buf[...]
        out_ref[out_rows(my_i, 1), :] = redq_buf[...]

        d[8].wait_recv()
        d[14].start()
        d[9].wait_recv()
        d[15].start()
        out_ref[out_rows(p1, 1), :] = s3q_buf[...]
        d[6].wait_recv()
        d[12].start()
        d[7].wait_recv()
        d[13].start()
        out_ref[out_rows(p2, 0), :] = s3p_buf[...]
        d[10].wait_recv()
        out_ref[out_rows(p1, 0), :] = s4p_buf[0]
        d[11].wait_recv()
        out_ref[out_rows(p2, 1), :] = s4q_buf[0]
        d[12].wait_recv()
        d[13].wait_recv()
        out_ref[out_rows(diag, 0), :] = s4p_buf[1]
        d[14].wait_recv()
        d[15].wait_recv()
        out_ref[out_rows(diag, 1), :] = s4q_buf[1]

        for k in range(len(d)):
            d[k].wait_send()

    out2d = pl.pallas_call(
        body,
        out_shape=jax.ShapeDtypeStruct((BS, D), jnp.float32),
        in_specs=[pl.BlockSpec(memory_space=pltpu.VMEM)] * 5,
        out_specs=pl.BlockSpec(memory_space=pltpu.VMEM),
        scratch_shapes=[
            pltpu.VMEM((N_DEV, 2, HSQ, D), jnp.float32),
            pltpu.VMEM((2, HSQ, D), jnp.float32),
            pltpu.VMEM((2, HSQ, D), jnp.float32),
            pltpu.VMEM((HSQ, D), jnp.float32),
            pltpu.VMEM((HSQ, D), jnp.float32),
            pltpu.VMEM((HSQ, D), jnp.float32),
            pltpu.VMEM((HSQ, D), jnp.float32),
            pltpu.VMEM((HSQ, D), jnp.float32),
            pltpu.VMEM((HSQ, D), jnp.float32),
            pltpu.VMEM((HSQ, D), jnp.float32),
            pltpu.VMEM((HSQ, D), jnp.float32),
            pltpu.VMEM((2, HSQ, D), jnp.float32),
            pltpu.VMEM((2, HSQ, D), jnp.float32),
            pltpu.SemaphoreType.DMA((16,)),
            pltpu.SemaphoreType.DMA((16,)),
        ],
        compiler_params=pltpu.CompilerParams(collective_id=0),
    )(x, Wq, Wo, Wk_sh, Wv_sh)
    return out2d.reshape(B, SQ, D)


# device time: 36743 ns/iter; 1.4533x vs baseline; 1.4533x over previous
import jax
import jax.numpy as jnp
from jax import lax
from jax.experimental import pallas as pl
from jax.experimental.pallas import tpu as pltpu

N_DEV = 4
B, SQ, D = 4, 256, 1024
HQ_SH = 8
HKV_SH = 2
DH = 128
GROUP = 4
SCALE = 0.08838834764831843
BS = B * SQ
HSQ = SQ // 2
QSQ = HSQ // 2


def kernel(x, Wq, Wo, Wk, Wv):
    kv_off = lax.axis_index("i") * (HKV_SH * DH)
    Wk_sh = lax.dynamic_slice_in_dim(Wk, kv_off, HKV_SH * DH, axis=1)
    Wv_sh = lax.dynamic_slice_in_dim(Wv, kv_off, HKV_SH * DH, axis=1)

    def body(x_ref, wq_ref, wo_ref, wk_ref, wv_ref, out_ref,
             pbuf, s1p_buf, s1q_buf, s2p_send, s2p_buf, s2q_send, s2q_buf,
             redp_buf, redq_buf, s3p_buf, s3q_buf, s4p_buf, s4q_buf,
             send_sems, recv_sems):
        my_i = lax.axis_index("i")
        p1 = my_i ^ 1
        p2 = 3 - my_i
        diag = (my_i + 2) % N_DEV

        barrier_sem = pltpu.get_barrier_semaphore()
        for o in range(1, N_DEV):
            pl.semaphore_signal(barrier_sem, inc=1,
                                device_id=((my_i + o) % N_DEV,),
                                device_id_type=pl.DeviceIdType.MESH)
        pl.semaphore_wait(barrier_sem, N_DEV - 1)

        q1 = pl.ds(0, QSQ)
        q2 = pl.ds(QSQ, QSQ)

        transfers = [
            (pbuf.at[0, 0], s1p_buf.at[0], p1),
            (pbuf.at[0, 1], s1q_buf.at[0], p2),
            (pbuf.at[1, 0], s1p_buf.at[1], p1),
            (pbuf.at[2, 1], s1q_buf.at[1], p2),
            (s2p_send, s2p_buf, p2),
            (s2q_send, s2q_buf, p1),
            (redp_buf.at[q1], s3p_buf.at[q1], p2),
            (redp_buf.at[q2], s3p_buf.at[q2], p2),
            (redq_buf.at[q1], s3q_buf.at[q1], p1),
            (redq_buf.at[q2], s3q_buf.at[q2], p1),
            (redp_buf, s4p_buf.at[0], p1),
            (redq_buf, s4q_buf.at[0], p2),
            (s3p_buf.at[q1], s4p_buf.at[1, q1], p1),
            (s3p_buf.at[q2], s4p_buf.at[1, q2], p1),
            (s3q_buf.at[q1], s4q_buf.at[1, q1], p2),
            (s3q_buf.at[q2], s4q_buf.at[1, q2], p2),
        ]
        d = [
            pltpu.make_async_remote_copy(
                src_ref=src, dst_ref=dst,
                send_sem=send_sems.at[k], recv_sem=recv_sems.at[k],
                device_id=(tgt,), device_id_type=pl.DeviceIdType.MESH,
            )
            for k, (src, dst, tgt) in enumerate(transfers)
        ]

        wq = wq_ref[...]
        wk = wk_ref[...]
        wv = wv_ref[...]
        wo = wo_ref[...]

        def store_partial(c, b):
            xb = x_ref[b]
            qb = jnp.dot(xb, wq, preferred_element_type=jnp.float32)
            kb = jnp.dot(xb, wk, preferred_element_type=jnp.float32)
            vb = jnp.dot(xb, wv, preferred_element_type=jnp.float32)
            outs = []
            for h in range(HQ_SH):
                g = h // GROUP
                q = qb[:, h * DH:(h + 1) * DH]
                k = kb[:, g * DH:(g + 1) * DH]
                v = vb[:, g * DH:(g + 1) * DH]
                s = jnp.dot(q, k.T, preferred_element_type=jnp.float32) * SCALE
                m = jnp.max(s, axis=-1, keepdims=True)
                p = jnp.exp(s - m)
                l = jnp.sum(p, axis=-1, keepdims=True)
                outs.append(jnp.dot(p, v, preferred_element_type=jnp.float32) / l)
            attn_b = jnp.concatenate(outs, axis=1)
            partial = jnp.dot(attn_b, wo, preferred_element_type=jnp.float32)
            pbuf[c, 0] = partial[:HSQ, :].astype(jnp.bfloat16)
            pbuf[c, 1] = partial[HSQ:, :].astype(jnp.bfloat16)

        def out_rows(b, half):
            return pl.ds(b * SQ + half * HSQ, HSQ)

        store_partial(0, diag)
        d[0].start()
        d[1].start()
        store_partial(1, p1)
        d[2].start()
        store_partial(2, p2)
        d[3].start()

        f32 = jnp.float32
        d[0].wait_recv()
        s2p_send[...] = (pbuf[2, 0].astype(f32)
                         + s1p_buf[0].astype(f32)).astype(jnp.bfloat16)
        d[4].start()
        d[1].wait_recv()
        s2q_send[...] = (pbuf[1, 1].astype(f32)
                         + s1q_buf[0].astype(f32)).astype(jnp.bfloat16)
        d[5].start()

        store_partial(3, my_i)

        d[3].wait_recv()
        d[5].wait_recv()
        redq_buf[...] = (pbuf[3, 1].astype(f32) + s1q_buf[1].astype(f32)
                         + s2q_buf[...].astype(f32)).astype(jnp.bfloat16)
        d[8].start()
        d[9].start()
        d[11].start()
        d[2].wait_recv()
        d[4].wait_recv()
        redp_buf[...] = (pbuf[3, 0].astype(f32) + s1p_buf[1].astype(f32)
                         + s2p_buf[...].astype(f32)).astype(jnp.bfloat16)
        d[6].start()
        d[7].start()
        d[10].start()
        out_ref[out_rows(my_i, 0), :] = redp_buf[...].astype(f32)
        out_ref[out_rows(my_i, 1), :] = redq_buf[...].astype(f32)

        d[8].wait_recv()
        d[14].start()
        d[9].wait_recv()
        d[15].start()
        out_ref[out_rows(p1, 1), :] = s3q_buf[...].astype(f32)
        d[6].wait_recv()
        d[12].start()
        d[7].wait_recv()
        d[13].start()
        out_ref[out_rows(p2, 0), :] = s3p_buf[...].astype(f32)
        d[10].wait_recv()
        out_ref[out_rows(p1, 0), :] = s4p_buf[0].astype(f32)
        d[11].wait_recv()
        out_ref[out_rows(p2, 1), :] = s4q_buf[0].astype(f32)
        d[12].wait_recv()
        d[13].wait_recv()
        out_ref[out_rows(diag, 0), :] = s4p_buf[1].astype(f32)
        d[14].wait_recv()
        d[15].wait_recv()
        out_ref[out_rows(diag, 1), :] = s4q_buf[1].astype(f32)

        for k in range(len(d)):
            d[k].wait_send()

    out2d = pl.pallas_call(
        body,
        out_shape=jax.ShapeDtypeStruct((BS, D), jnp.float32),
        in_specs=[pl.BlockSpec(memory_space=pltpu.VMEM)] * 5,
        out_specs=pl.BlockSpec(memory_space=pltpu.VMEM),
        scratch_shapes=[
            pltpu.VMEM((N_DEV, 2, HSQ, D), jnp.bfloat16),
            pltpu.VMEM((2, HSQ, D), jnp.bfloat16),
            pltpu.VMEM((2, HSQ, D), jnp.bfloat16),
            pltpu.VMEM((HSQ, D), jnp.bfloat16),
            pltpu.VMEM((HSQ, D), jnp.bfloat16),
            pltpu.VMEM((HSQ, D), jnp.bfloat16),
            pltpu.VMEM((HSQ, D), jnp.bfloat16),
            pltpu.VMEM((HSQ, D), jnp.bfloat16),
            pltpu.VMEM((HSQ, D), jnp.bfloat16),
            pltpu.VMEM((HSQ, D), jnp.bfloat16),
            pltpu.VMEM((HSQ, D), jnp.bfloat16),
            pltpu.VMEM((2, HSQ, D), jnp.bfloat16),
            pltpu.VMEM((2, HSQ, D), jnp.bfloat16),
            pltpu.SemaphoreType.DMA((16,)),
            pltpu.SemaphoreType.DMA((16,)),
        ],
        compiler_params=pltpu.CompilerParams(collective_id=0),
    )(x, Wq, Wo, Wk_sh, Wv_sh)
    return out2d.reshape(B, SQ, D)
